# baseline (speedup 1.0000x reference)
import functools

import jax
import jax.numpy as jnp
from jax import lax
from jax.experimental import pallas as pl
from jax.experimental.pallas import tpu as pltpu
from jax.experimental.pallas import tpu_sc as plsc

B, A, M = 64, 128, 12
ORIG, NBR, AF, NC = 92, 41, 64, 3
N = B * A
K = N * M
F2 = 2 * AF

_SC_CORES, _SC_SUBCORES = 2, 16
_NW = _SC_CORES * _SC_SUBCORES
_ROWS_PER_W = K // _NW
_CH = 128
_NCH = _ROWS_PER_W // _CH


def _sc_gather(table, idx, nrows=K):
    mesh = plsc.VectorSubcoreMesh(core_axis_name="c", subcore_axis_name="s")
    rows_per_w = nrows // _NW
    nch = rows_per_w // _CH

    @functools.partial(
        pl.kernel, mesh=mesh,
        out_type=jax.ShapeDtypeStruct((nrows, F2), jnp.float32),
        scratch_types=[
            pltpu.VMEM((rows_per_w,), jnp.int32),
            pltpu.VMEM((_CH, F2), jnp.float32),
            pltpu.VMEM((_CH, F2), jnp.float32),
            pltpu.SemaphoreType.DMA,
            pltpu.SemaphoreType.DMA,
        ],
    )
    def gk(table_hbm, idx_hbm, out_hbm, idx_all, rows0, rows1, sem0, sem1):
        wid = lax.axis_index("s") * _SC_CORES + lax.axis_index("c")
        base_w = pl.multiple_of(wid * rows_per_w, rows_per_w)
        rows = (rows0, rows1)
        sems = (sem0, sem1)
        pltpu.sync_copy(idx_hbm.at[pl.ds(base_w, rows_per_w)], idx_all)
        for b in range(2):
            pltpu.async_copy(
                table_hbm.at[idx_all.at[pl.ds(b * _CH, _CH)]], rows[b], sems[b])

        def body(it, carry):
            for b in range(2):
                j = 2 * it + b
                pltpu.make_async_copy(
                    table_hbm.at[idx_all.at[pl.ds(0, _CH)]], rows[b], sems[b]).wait()
                pltpu.sync_copy(
                    rows[b], out_hbm.at[pl.ds(base_w + j * _CH, _CH)])
                j2 = j + 2

                @pl.when(j2 < nch)
                def _():
                    pltpu.async_copy(
                        table_hbm.at[idx_all.at[pl.ds(j2 * _CH, _CH)]],
                        rows[b], sems[b])
            return carry

        lax.fori_loop(0, nch // 2, body, 0)

    return gk(table, idx)


def _softplus(z):
    return jnp.log(1.0 + jnp.exp(-jnp.abs(z))) + jnp.maximum(z, 0.0)


def _sigmoid(z):
    return 0.5 * jnp.tanh(0.5 * z) + 0.5


_RT = 1024
_NT = 1024
_GT = _NT * M


def _embed_body(af_ref, wemb_ref, wcat_ref, x_ref, ps_ref, pn_ref):
    x = jnp.dot(af_ref[...], wemb_ref[...], preferred_element_type=jnp.float32)
    x_ref[...] = x
    p = jnp.dot(x, wcat_ref[...], preferred_element_type=jnp.float32)
    ps_ref[...] = p[:, :F2]
    pn_ref[...] = p[:, F2:]


def _embed(atom_fea, W_emb, Wcat):
    return pl.pallas_call(
        _embed_body,
        grid=(N // _RT,),
        in_specs=[
            pl.BlockSpec((_RT, ORIG), lambda i: (i, 0)),
            pl.BlockSpec((ORIG, AF), lambda i: (0, 0)),
            pl.BlockSpec((AF, 2 * F2), lambda i: (0, 0)),
        ],
        out_specs=[
            pl.BlockSpec((_RT, AF), lambda i: (i, 0)),
            pl.BlockSpec((_RT, F2), lambda i: (i, 0)),
            pl.BlockSpec((_RT, F2), lambda i: (i, 0)),
        ],
        out_shape=[
            jax.ShapeDtypeStruct((N, AF), jnp.float32),
            jax.ShapeDtypeStruct((N, F2), jnp.float32),
            jax.ShapeDtypeStruct((N, F2), jnp.float32),
        ],
    )(atom_fea, W_emb, Wcat)


_MH = M // 2


def _stats_body(an_ref, nbr_ref, ps_ref, wfe_ref, g_ref, su_ref, sq_ref):
    nbr = nbr_ref[...].reshape(_MH * _NT, NBR)
    pe = jnp.dot(nbr, wfe_ref[...], preferred_element_type=jnp.float32)
    psr = jnp.broadcast_to(ps_ref[...][None], (_MH, _NT, F2)).reshape(_MH * _NT, F2)
    g = pe + an_ref[...].reshape(_MH * _NT, F2) + psr
    g_ref[...] = g.astype(jnp.bfloat16).reshape(_MH, _NT, F2)

    @pl.when(pl.program_id(0) == 0)
    def _():
        su_ref[...] = jnp.zeros_like(su_ref)
        sq_ref[...] = jnp.zeros_like(sq_ref)

    su_ref[...] += jnp.sum(g, axis=0, keepdims=True)
    sq_ref[...] += jnp.sum(g * g, axis=0, keepdims=True)


def _stats(an3, nbr3h, ps, Wfe):
    return pl.pallas_call(
        _stats_body,
        grid=(N // _NT,),
        in_specs=[
            pl.BlockSpec((_MH, _NT, F2), lambda i: (0, i, 0)),
            pl.BlockSpec((_MH, _NT, NBR), lambda i: (0, i, 0)),
            pl.BlockSpec((_NT, F2), lambda i: (i, 0)),
            pl.BlockSpec((NBR, F2), lambda i: (0, 0)),
        ],
        out_specs=[
            pl.BlockSpec((_MH, _NT, F2), lambda i: (0, i, 0)),
            pl.BlockSpec((1, F2), lambda i: (0, 0)),
            pl.BlockSpec((1, F2), lambda i: (0, 0)),
        ],
        out_shape=[
            jax.ShapeDtypeStruct((_MH, N, F2), jnp.bfloat16),
            jax.ShapeDtypeStruct((1, F2), jnp.float32),
            jax.ShapeDtypeStruct((1, F2), jnp.float32),
        ],
    )(an3, nbr3h, ps, Wfe)


def _act_body(ga_ref, gb_ref, su_ref, sq_ref, g1_ref, bt1_ref, s_ref, ssu_ref, ssq_ref):
    mean = su_ref[...] / K
    var = sq_ref[...] / K - mean * mean
    a = g1_ref[...] * lax.rsqrt(var + 1e-5)
    c = bt1_ref[...] - mean * a
    s = None
    for g_ref in (ga_ref, gb_ref):
        g = g_ref[...].astype(jnp.float32).reshape(_MH * _NT, F2)
        gh = g * a + c
        fl = _sigmoid(gh[:, :AF])
        co = _softplus(gh[:, AF:])
        prod = (fl * co).reshape(_MH, _NT, AF)
        for m in range(_MH):
            s = prod[m] if s is None else s + prod[m]
    s_ref[...] = s

    @pl.when(pl.program_id(0) == 0)
    def _():
        ssu_ref[...] = jnp.zeros_like(ssu_ref)
        ssq_ref[...] = jnp.zeros_like(ssq_ref)

    ssu_ref[...] += jnp.sum(s, axis=0, keepdims=True)
    ssq_ref[...] += jnp.sum(s * s, axis=0, keepdims=True)


def _act(ga, gb, su, sq, g1, bt1):
    return pl.pallas_call(
        _act_body,
        grid=(N // _NT,),
        in_specs=[
            pl.BlockSpec((_MH, _NT, F2), lambda i: (0, i, 0)),
            pl.BlockSpec((_MH, _NT, F2), lambda i: (0, i, 0)),
            pl.BlockSpec((1, F2), lambda i: (0, 0)),
            pl.BlockSpec((1, F2), lambda i: (0, 0)),
            pl.BlockSpec((1, F2), lambda i: (0, 0)),
            pl.BlockSpec((1, F2), lambda i: (0, 0)),
        ],
        out_specs=[
            pl.BlockSpec((_NT, AF), lambda i: (i, 0)),
            pl.BlockSpec((1, AF), lambda i: (0, 0)),
            pl.BlockSpec((1, AF), lambda i: (0, 0)),
        ],
        out_shape=[
            jax.ShapeDtypeStruct((N, AF), jnp.float32),
            jax.ShapeDtypeStruct((1, AF), jnp.float32),
            jax.ShapeDtypeStruct((1, AF), jnp.float32),
        ],
    )(ga, gb, su, sq, g1, bt1)


def _bn2_update(x_ref, s_ref, ssu_ref, ssq_ref, g2_ref, bt2_ref):
    m2 = ssu_ref[...] / N
    v2 = ssq_ref[...] / N - m2 * m2
    a2 = g2_ref[...] * lax.rsqrt(v2 + 1e-5)
    c2 = bt2_ref[...] - m2 * a2
    return _softplus(x_ref[...] + s_ref[...] * a2 + c2)


def _update_body(x_ref, s_ref, ssu_ref, ssq_ref, g2_ref, bt2_ref, wcat_ref,
                 xo_ref, ps_ref, pn_ref):
    xn = _bn2_update(x_ref, s_ref, ssu_ref, ssq_ref, g2_ref, bt2_ref)
    xo_ref[...] = xn
    p = jnp.dot(xn, wcat_ref[...], preferred_element_type=jnp.float32)
    ps_ref[...] = p[:, :F2]
    pn_ref[...] = p[:, F2:]


def _update(x, s, ssu, ssq, g2, bt2, Wcat):
    return pl.pallas_call(
        _update_body,
        grid=(N // _RT,),
        in_specs=[
            pl.BlockSpec((_RT, AF), lambda i: (i, 0)),
            pl.BlockSpec((_RT, AF), lambda i: (i, 0)),
            pl.BlockSpec((1, AF), lambda i: (0, 0)),
            pl.BlockSpec((1, AF), lambda i: (0, 0)),
            pl.BlockSpec((1, AF), lambda i: (0, 0)),
            pl.BlockSpec((1, AF), lambda i: (0, 0)),
            pl.BlockSpec((AF, 2 * F2), lambda i: (0, 0)),
        ],
        out_specs=[
            pl.BlockSpec((_RT, AF), lambda i: (i, 0)),
            pl.BlockSpec((_RT, F2), lambda i: (i, 0)),
            pl.BlockSpec((_RT, F2), lambda i: (i, 0)),
        ],
        out_shape=[
            jax.ShapeDtypeStruct((N, AF), jnp.float32),
            jax.ShapeDtypeStruct((N, F2), jnp.float32),
            jax.ShapeDtypeStruct((N, F2), jnp.float32),
        ],
    )(x, s, ssu, ssq, g2, bt2, Wcat)


def _update_last_body(x_ref, s_ref, ssu_ref, ssq_ref, g2_ref, bt2_ref, xo_ref):
    xo_ref[...] = _bn2_update(x_ref, s_ref, ssu_ref, ssq_ref, g2_ref, bt2_ref)


def _update_last(x, s, ssu, ssq, g2, bt2):
    return pl.pallas_call(
        _update_last_body,
        grid=(N // _RT,),
        in_specs=[
            pl.BlockSpec((_RT, AF), lambda i: (i, 0)),
            pl.BlockSpec((_RT, AF), lambda i: (i, 0)),
            pl.BlockSpec((1, AF), lambda i: (0, 0)),
            pl.BlockSpec((1, AF), lambda i: (0, 0)),
            pl.BlockSpec((1, AF), lambda i: (0, 0)),
            pl.BlockSpec((1, AF), lambda i: (0, 0)),
        ],
        out_specs=pl.BlockSpec((_RT, AF), lambda i: (i, 0)),
        out_shape=jax.ShapeDtypeStruct((N, AF), jnp.float32),
    )(x, s, ssu, ssq, g2, bt2)


def _final_body(x_ref, w2c_ref, b2_ref, waf_ref, baf_ref,
                ep_ref, af_ref, z_ref, n_ref):
    x = x_ref[...]
    nrm = jnp.sqrt(jnp.sum(x * x, axis=1, keepdims=True))
    nd = x / jnp.maximum(nrm, 1e-12)
    n_ref[...] = nd[None]
    z_ref[...] = jnp.mean(nd, axis=0, keepdims=True)[None]
    af_ref[...] = jnp.dot(nd, waf_ref[...], preferred_element_type=jnp.float32) + baf_ref[...]
    tmp = jnp.dot(nd, w2c_ref[...], preferred_element_type=jnp.float32)
    tmp_r = jnp.concatenate([tmp[:, j * AF:(j + 1) * AF] for j in range(6)], axis=0)
    es_all = lax.dot_general(tmp_r, nd, (((1,), (1,)), ((), ())),
                             preferred_element_type=jnp.float32)
    es = [es_all[j * A:(j + 1) * A, :] + b2_ref[0, j] for j in range(6)]
    mx = es[0]
    for j in range(1, 6):
        mx = jnp.maximum(mx, es[j])
    se = jnp.exp(es[0] - mx)
    for j in range(1, 6):
        se += jnp.exp(es[j] - mx)
    off = mx + jnp.log(se)
    ep_ref[...] = jnp.concatenate([es[j] - off for j in range(6)], axis=0)[None]


def _final(x, W2c, b2, W_af, b_af):
    return pl.pallas_call(
        _final_body,
        grid=(B,),
        in_specs=[
            pl.BlockSpec((A, AF), lambda i: (i, 0)),
            pl.BlockSpec((AF, 6 * AF), lambda i: (0, 0)),
            pl.BlockSpec((1, 6), lambda i: (0, 0)),
            pl.BlockSpec((AF, ORIG), lambda i: (0, 0)),
            pl.BlockSpec((1, ORIG), lambda i: (0, 0)),
        ],
        out_specs=[
            pl.BlockSpec((1, 6 * A, A), lambda i: (i, 0, 0)),
            pl.BlockSpec((A, ORIG), lambda i: (i, 0)),
            pl.BlockSpec((1, 1, AF), lambda i: (i, 0, 0)),
            pl.BlockSpec((1, A, AF), lambda i: (i, 0, 0)),
        ],
        out_shape=[
            jax.ShapeDtypeStruct((B, 6 * A, A), jnp.float32),
            jax.ShapeDtypeStruct((N, ORIG), jnp.float32),
            jax.ShapeDtypeStruct((B, 1, AF), jnp.float32),
            jax.ShapeDtypeStruct((B, A, AF), jnp.float32),
        ],
    )(x, W2c, b2, W_af, b_af)


def kernel(atom_fea, nbr_fea, nbr_fea_idx, crystal_atom_idx, cuda_flag, W_emb,
           Wf0, bf0, g1_0, bt1_0, g2_0, bt2_0,
           Wf1, bf1, g1_1, bt1_1, g2_1, bt2_1,
           Wf2, bf2, g1_2, bt1_2, g2_2, bt2_2,
           W_bil, b_bil, W_fc1, b_fc1, W_af, b_af):
    Wf = [Wf0, Wf1, Wf2]
    g1 = [g1_0[None], g1_1[None], g1_2[None]]
    bt1 = [bt1_0[None], bt1_1[None], bt1_2[None]]
    g2 = [g2_0[None], g2_1[None], g2_2[None]]
    bt2 = [bt2_0[None], bt2_1[None], bt2_2[None]]
    Wcat = [jnp.concatenate([w[:AF], w[AF:2 * AF]], axis=1) for w in Wf]
    Wfe = [w[2 * AF:].astype(jnp.bfloat16) for w in Wf]
    idx = nbr_fea_idx.T.reshape(-1).astype(jnp.int32)
    idx_a, idx_b = idx[:K // 2], idx[K // 2:]
    nbr3 = nbr_fea.transpose(1, 0, 2).astype(jnp.bfloat16)
    nbr_a, nbr_b = nbr3[:_MH], nbr3[_MH:]

    x, ps, pn = _embed(atom_fea, W_emb, Wcat[0])
    for l in range(NC):
        an_a = _sc_gather(pn, idx_a, K // 2).reshape(_MH, N, F2)
        an_b = _sc_gather(pn, idx_b, K // 2).reshape(_MH, N, F2)
        ga, sua, sqa = _stats(an_a, nbr_a, ps, Wfe[l])
        gb, sub, sqb = _stats(an_b, nbr_b, ps, Wfe[l])
        su = sua + sub
        sq = sqa + sqb
        s, ssu, ssq = _act(ga, gb, su, sq, g1[l], bt1[l])
        if l + 1 < NC:
            x, ps, pn = _update(x, s, ssu, ssq, g2[l], bt2[l], Wcat[l + 1])
        else:
            x = _update_last(x, s, ssu, ssq, g2[l], bt2[l])

    W2 = jnp.einsum('kde,kj->jde', W_bil, W_fc1)
    W2c = jnp.concatenate([W2[j] for j in range(6)], axis=1)
    b2 = (b_bil @ W_fc1 + b_fc1)[None]
    epk, af, z, normed = _final(x, W2c, b2, W_af, b_af[None])
    ep = jnp.transpose(epk.reshape(B, 6, A, A), (0, 2, 3, 1)).reshape(-1, 6)
    return ep, af, z.reshape(B, AF), normed, x


# speedup vs baseline: 1.0574x; 1.0574x over previous
import functools

import jax
import jax.numpy as jnp
from jax import lax
from jax.experimental import pallas as pl
from jax.experimental.pallas import tpu as pltpu
from jax.experimental.pallas import tpu_sc as plsc

B, A, M = 64, 128, 12
ORIG, NBR, AF, NC = 92, 41, 64, 3
N = B * A
K = N * M
F2 = 2 * AF

_SC_CORES, _SC_SUBCORES = 2, 16
_NW = _SC_CORES * _SC_SUBCORES
_ROWS_PER_W = K // _NW
_CH = 128
_NCH = _ROWS_PER_W // _CH


def _sc_gather(table, idx, nrows=K):
    mesh = plsc.VectorSubcoreMesh(core_axis_name="c", subcore_axis_name="s")
    rows_per_w = nrows // _NW
    nch = rows_per_w // _CH

    @functools.partial(
        pl.kernel, mesh=mesh,
        out_type=jax.ShapeDtypeStruct((nrows, F2), jnp.float32),
        scratch_types=[
            pltpu.VMEM((rows_per_w,), jnp.int32),
            pltpu.VMEM((_CH, F2), jnp.float32),
            pltpu.VMEM((_CH, F2), jnp.float32),
            pltpu.SemaphoreType.DMA,
            pltpu.SemaphoreType.DMA,
        ],
    )
    def gk(table_hbm, idx_hbm, out_hbm, idx_all, rows0, rows1, sem0, sem1):
        wid = lax.axis_index("s") * _SC_CORES + lax.axis_index("c")
        base_w = pl.multiple_of(wid * rows_per_w, rows_per_w)
        rows = (rows0, rows1)
        sems = (sem0, sem1)
        pltpu.sync_copy(idx_hbm.at[pl.ds(base_w, rows_per_w)], idx_all)
        for b in range(2):
            pltpu.async_copy(
                table_hbm.at[idx_all.at[pl.ds(b * _CH, _CH)]], rows[b], sems[b])

        def body(it, carry):
            for b in range(2):
                j = 2 * it + b
                pltpu.make_async_copy(
                    table_hbm.at[idx_all.at[pl.ds(0, _CH)]], rows[b], sems[b]).wait()
                pltpu.sync_copy(
                    rows[b], out_hbm.at[pl.ds(base_w + j * _CH, _CH)])
                j2 = j + 2

                @pl.when(j2 < nch)
                def _():
                    pltpu.async_copy(
                        table_hbm.at[idx_all.at[pl.ds(j2 * _CH, _CH)]],
                        rows[b], sems[b])
            return carry

        lax.fori_loop(0, nch // 2, body, 0)

    return gk(table, idx)


def _softplus(z):
    return jnp.log(1.0 + jnp.exp(-jnp.abs(z))) + jnp.maximum(z, 0.0)


def _sigmoid(z):
    return 0.5 * jnp.tanh(0.5 * z) + 0.5


_RT = 1024
_NT = 1024
_GT = _NT * M


def _embed_body(af_ref, wemb_ref, wcat_ref, x_ref, ps_ref, pn_ref):
    x = jnp.dot(af_ref[...], wemb_ref[...], preferred_element_type=jnp.float32)
    x_ref[...] = x
    p = jnp.dot(x, wcat_ref[...], preferred_element_type=jnp.float32)
    ps_ref[...] = p[:, :F2]
    pn_ref[...] = p[:, F2:]


def _embed(atom_fea, W_emb, Wcat):
    return pl.pallas_call(
        _embed_body,
        grid=(N // _RT,),
        in_specs=[
            pl.BlockSpec((_RT, ORIG), lambda i: (i, 0)),
            pl.BlockSpec((ORIG, AF), lambda i: (0, 0)),
            pl.BlockSpec((AF, 2 * F2), lambda i: (0, 0)),
        ],
        out_specs=[
            pl.BlockSpec((_RT, AF), lambda i: (i, 0)),
            pl.BlockSpec((_RT, F2), lambda i: (i, 0)),
            pl.BlockSpec((_RT, F2), lambda i: (i, 0)),
        ],
        out_shape=[
            jax.ShapeDtypeStruct((N, AF), jnp.float32),
            jax.ShapeDtypeStruct((N, F2), jnp.float32),
            jax.ShapeDtypeStruct((N, F2), jnp.float32),
        ],
    )(atom_fea, W_emb, Wcat)


def _stats_body(an_ref, nbr_ref, ps_ref, wfe_ref, g_ref, su_ref, sq_ref):
    nbr = nbr_ref[...].reshape(M * _NT, NBR)
    pe = jnp.dot(nbr, wfe_ref[...], preferred_element_type=jnp.float32)
    psr = jnp.broadcast_to(ps_ref[...][None], (M, _NT, F2)).reshape(M * _NT, F2)
    g = pe + an_ref[...].reshape(M * _NT, F2) + psr
    g_ref[...] = g.astype(jnp.bfloat16).reshape(M, _NT, F2)

    @pl.when(pl.program_id(0) == 0)
    def _():
        su_ref[...] = jnp.zeros_like(su_ref)
        sq_ref[...] = jnp.zeros_like(sq_ref)

    su_ref[...] += jnp.sum(g, axis=0, keepdims=True)
    sq_ref[...] += jnp.sum(g * g, axis=0, keepdims=True)


def _stats(an3, nbr3, ps, Wfe):
    return pl.pallas_call(
        _stats_body,
        grid=(N // _NT,),
        in_specs=[
            pl.BlockSpec((M, _NT, F2), lambda i: (0, i, 0)),
            pl.BlockSpec((M, _NT, NBR), lambda i: (0, i, 0)),
            pl.BlockSpec((_NT, F2), lambda i: (i, 0)),
            pl.BlockSpec((NBR, F2), lambda i: (0, 0)),
        ],
        out_specs=[
            pl.BlockSpec((M, _NT, F2), lambda i: (0, i, 0)),
            pl.BlockSpec((1, F2), lambda i: (0, 0)),
            pl.BlockSpec((1, F2), lambda i: (0, 0)),
        ],
        out_shape=[
            jax.ShapeDtypeStruct((M, N, F2), jnp.bfloat16),
            jax.ShapeDtypeStruct((1, F2), jnp.float32),
            jax.ShapeDtypeStruct((1, F2), jnp.float32),
        ],
    )(an3, nbr3, ps, Wfe)


def _act_body(g_ref, su_ref, sq_ref, g1_ref, bt1_ref, s_ref, ssu_ref, ssq_ref):
    mean = su_ref[...] / K
    var = sq_ref[...] / K - mean * mean
    a = g1_ref[...] * lax.rsqrt(var + 1e-5)
    c = bt1_ref[...] - mean * a
    g = g_ref[...].astype(jnp.float32).reshape(M * _NT, F2)
    gh = g * a + c
    fl = _sigmoid(gh[:, :AF])
    co = _softplus(gh[:, AF:])
    prod = (fl * co).reshape(M, _NT, AF)
    s = prod[0]
    for m in range(1, M):
        s = s + prod[m]
    s_ref[...] = s

    @pl.when(pl.program_id(0) == 0)
    def _():
        ssu_ref[...] = jnp.zeros_like(ssu_ref)
        ssq_ref[...] = jnp.zeros_like(ssq_ref)

    ssu_ref[...] += jnp.sum(s, axis=0, keepdims=True)
    ssq_ref[...] += jnp.sum(s * s, axis=0, keepdims=True)


def _act(g, su, sq, g1, bt1):
    return pl.pallas_call(
        _act_body,
        grid=(N // _NT,),
        in_specs=[
            pl.BlockSpec((M, _NT, F2), lambda i: (0, i, 0)),
            pl.BlockSpec((1, F2), lambda i: (0, 0)),
            pl.BlockSpec((1, F2), lambda i: (0, 0)),
            pl.BlockSpec((1, F2), lambda i: (0, 0)),
            pl.BlockSpec((1, F2), lambda i: (0, 0)),
        ],
        out_specs=[
            pl.BlockSpec((_NT, AF), lambda i: (i, 0)),
            pl.BlockSpec((1, AF), lambda i: (0, 0)),
            pl.BlockSpec((1, AF), lambda i: (0, 0)),
        ],
        out_shape=[
            jax.ShapeDtypeStruct((N, AF), jnp.float32),
            jax.ShapeDtypeStruct((1, AF), jnp.float32),
            jax.ShapeDtypeStruct((1, AF), jnp.float32),
        ],
    )(g, su, sq, g1, bt1)


def _bn2_update(x_ref, s_ref, ssu_ref, ssq_ref, g2_ref, bt2_ref):
    m2 = ssu_ref[...] / N
    v2 = ssq_ref[...] / N - m2 * m2
    a2 = g2_ref[...] * lax.rsqrt(v2 + 1e-5)
    c2 = bt2_ref[...] - m2 * a2
    return _softplus(x_ref[...] + s_ref[...] * a2 + c2)


def _update_body(x_ref, s_ref, ssu_ref, ssq_ref, g2_ref, bt2_ref, wcat_ref,
                 xo_ref, ps_ref, pn_ref):
    xn = _bn2_update(x_ref, s_ref, ssu_ref, ssq_ref, g2_ref, bt2_ref)
    xo_ref[...] = xn
    p = jnp.dot(xn, wcat_ref[...], preferred_element_type=jnp.float32)
    ps_ref[...] = p[:, :F2]
    pn_ref[...] = p[:, F2:]


def _update(x, s, ssu, ssq, g2, bt2, Wcat):
    return pl.pallas_call(
        _update_body,
        grid=(N // _RT,),
        in_specs=[
            pl.BlockSpec((_RT, AF), lambda i: (i, 0)),
            pl.BlockSpec((_RT, AF), lambda i: (i, 0)),
            pl.BlockSpec((1, AF), lambda i: (0, 0)),
            pl.BlockSpec((1, AF), lambda i: (0, 0)),
            pl.BlockSpec((1, AF), lambda i: (0, 0)),
            pl.BlockSpec((1, AF), lambda i: (0, 0)),
            pl.BlockSpec((AF, 2 * F2), lambda i: (0, 0)),
        ],
        out_specs=[
            pl.BlockSpec((_RT, AF), lambda i: (i, 0)),
            pl.BlockSpec((_RT, F2), lambda i: (i, 0)),
            pl.BlockSpec((_RT, F2), lambda i: (i, 0)),
        ],
        out_shape=[
            jax.ShapeDtypeStruct((N, AF), jnp.float32),
            jax.ShapeDtypeStruct((N, F2), jnp.float32),
            jax.ShapeDtypeStruct((N, F2), jnp.float32),
        ],
    )(x, s, ssu, ssq, g2, bt2, Wcat)


def _update_last_body(x_ref, s_ref, ssu_ref, ssq_ref, g2_ref, bt2_ref, xo_ref):
    xo_ref[...] = _bn2_update(x_ref, s_ref, ssu_ref, ssq_ref, g2_ref, bt2_ref)


def _update_last(x, s, ssu, ssq, g2, bt2):
    return pl.pallas_call(
        _update_last_body,
        grid=(N // _RT,),
        in_specs=[
            pl.BlockSpec((_RT, AF), lambda i: (i, 0)),
            pl.BlockSpec((_RT, AF), lambda i: (i, 0)),
            pl.BlockSpec((1, AF), lambda i: (0, 0)),
            pl.BlockSpec((1, AF), lambda i: (0, 0)),
            pl.BlockSpec((1, AF), lambda i: (0, 0)),
            pl.BlockSpec((1, AF), lambda i: (0, 0)),
        ],
        out_specs=pl.BlockSpec((_RT, AF), lambda i: (i, 0)),
        out_shape=jax.ShapeDtypeStruct((N, AF), jnp.float32),
    )(x, s, ssu, ssq, g2, bt2)


def _final_body(x_ref, w2c_ref, b2_ref, waf_ref, baf_ref,
                ep_ref, af_ref, z_ref, n_ref):
    x = x_ref[...]
    nrm = jnp.sqrt(jnp.sum(x * x, axis=1, keepdims=True))
    nd = x / jnp.maximum(nrm, 1e-12)
    n_ref[...] = nd[None]
    z_ref[...] = jnp.mean(nd, axis=0, keepdims=True)[None]
    af_ref[...] = jnp.dot(nd, waf_ref[...], preferred_element_type=jnp.float32) + baf_ref[...]
    tmp = jnp.dot(nd, w2c_ref[...], preferred_element_type=jnp.float32)
    tmp_r = jnp.concatenate([tmp[:, j * AF:(j + 1) * AF] for j in range(6)], axis=0)
    es_all = lax.dot_general(tmp_r, nd, (((1,), (1,)), ((), ())),
                             preferred_element_type=jnp.float32)
    es = [es_all[j * A:(j + 1) * A, :] + b2_ref[0, j] for j in range(6)]
    mx = es[0]
    for j in range(1, 6):
        mx = jnp.maximum(mx, es[j])
    se = jnp.exp(es[0] - mx)
    for j in range(1, 6):
        se += jnp.exp(es[j] - mx)
    off = mx + jnp.log(se)
    ep_ref[...] = jnp.concatenate([es[j] - off for j in range(6)], axis=0)[None]


def _final(x, W2c, b2, W_af, b_af):
    return pl.pallas_call(
        _final_body,
        grid=(B,),
        in_specs=[
            pl.BlockSpec((A, AF), lambda i: (i, 0)),
            pl.BlockSpec((AF, 6 * AF), lambda i: (0, 0)),
            pl.BlockSpec((1, 6), lambda i: (0, 0)),
            pl.BlockSpec((AF, ORIG), lambda i: (0, 0)),
            pl.BlockSpec((1, ORIG), lambda i: (0, 0)),
        ],
        out_specs=[
            pl.BlockSpec((1, 6 * A, A), lambda i: (i, 0, 0)),
            pl.BlockSpec((A, ORIG), lambda i: (i, 0)),
            pl.BlockSpec((1, 1, AF), lambda i: (i, 0, 0)),
            pl.BlockSpec((1, A, AF), lambda i: (i, 0, 0)),
        ],
        out_shape=[
            jax.ShapeDtypeStruct((B, 6 * A, A), jnp.float32),
            jax.ShapeDtypeStruct((N, ORIG), jnp.float32),
            jax.ShapeDtypeStruct((B, 1, AF), jnp.float32),
            jax.ShapeDtypeStruct((B, A, AF), jnp.float32),
        ],
    )(x, W2c, b2, W_af, b_af)


def kernel(atom_fea, nbr_fea, nbr_fea_idx, crystal_atom_idx, cuda_flag, W_emb,
           Wf0, bf0, g1_0, bt1_0, g2_0, bt2_0,
           Wf1, bf1, g1_1, bt1_1, g2_1, bt2_1,
           Wf2, bf2, g1_2, bt1_2, g2_2, bt2_2,
           W_bil, b_bil, W_fc1, b_fc1, W_af, b_af):
    Wf = [Wf0, Wf1, Wf2]
    g1 = [g1_0[None], g1_1[None], g1_2[None]]
    bt1 = [bt1_0[None], bt1_1[None], bt1_2[None]]
    g2 = [g2_0[None], g2_1[None], g2_2[None]]
    bt2 = [bt2_0[None], bt2_1[None], bt2_2[None]]
    Wcat = [jnp.concatenate([w[:AF], w[AF:2 * AF]], axis=1) for w in Wf]
    Wfe = [w[2 * AF:].astype(jnp.bfloat16) for w in Wf]
    idx = nbr_fea_idx.T.reshape(-1).astype(jnp.int32)
    nbr3 = nbr_fea.transpose(1, 0, 2).astype(jnp.bfloat16)

    x, ps, pn = _embed(atom_fea, W_emb, Wcat[0])
    for l in range(NC):
        an3 = _sc_gather(pn, idx).reshape(M, N, F2)
        g, su, sq = _stats(an3, nbr3, ps, Wfe[l])
        s, ssu, ssq = _act(g, su, sq, g1[l], bt1[l])
        if l + 1 < NC:
            x, ps, pn = _update(x, s, ssu, ssq, g2[l], bt2[l], Wcat[l + 1])
        else:
            x = _update_last(x, s, ssu, ssq, g2[l], bt2[l])

    W2 = jnp.einsum('kde,kj->jde', W_bil, W_fc1)
    W2c = jnp.concatenate([W2[j] for j in range(6)], axis=1)
    b2 = (b_bil @ W_fc1 + b_fc1)[None]
    epk, af, z, normed = _final(x, W2c, b2, W_af, b_af[None])
    ep = jnp.transpose(epk.reshape(B, 6, A, A), (0, 2, 3, 1)).reshape(-1, 6)
    return ep, af, z.reshape(B, AF), normed, x


# speedup vs baseline: 1.0608x; 1.0032x over previous
import functools

import jax
import jax.numpy as jnp
from jax import lax
from jax.experimental import pallas as pl
from jax.experimental.pallas import tpu as pltpu
from jax.experimental.pallas import tpu_sc as plsc

B, A, M = 64, 128, 12
ORIG, NBR, AF, NC = 92, 41, 64, 3
N = B * A
K = N * M
F2 = 2 * AF

_SC_CORES, _SC_SUBCORES = 2, 16
_NW = _SC_CORES * _SC_SUBCORES
_ROWS_PER_W = K // _NW
_CH = 128
_NCH = _ROWS_PER_W // _CH


def _sc_gather(table, idx, nrows=K):
    mesh = plsc.VectorSubcoreMesh(core_axis_name="c", subcore_axis_name="s")
    rows_per_w = nrows // _NW
    nch = rows_per_w // _CH

    nbuf = 4

    @functools.partial(
        pl.kernel, mesh=mesh,
        out_type=jax.ShapeDtypeStruct((nrows, F2), jnp.float32),
        scratch_types=[
            pltpu.VMEM((rows_per_w,), jnp.int32),
        ] + [pltpu.VMEM((_CH, F2), jnp.float32) for _ in range(4)]
          + [pltpu.SemaphoreType.DMA for _ in range(8)],
    )
    def gk(table_hbm, idx_hbm, out_hbm, idx_all,
           rows0, rows1, rows2, rows3,
           g0, g1_, g2_, g3, w0, w1, w2, w3):
        wid = lax.axis_index("s") * _SC_CORES + lax.axis_index("c")
        base_w = pl.multiple_of(wid * rows_per_w, rows_per_w)
        rows = (rows0, rows1, rows2, rows3)
        gsem = (g0, g1_, g2_, g3)
        wsem = (w0, w1, w2, w3)
        pltpu.sync_copy(idx_hbm.at[pl.ds(base_w, rows_per_w)], idx_all)
        for b in range(nbuf):
            pltpu.async_copy(
                table_hbm.at[idx_all.at[pl.ds(b * _CH, _CH)]], rows[b], gsem[b])

        def body(it, carry):
            for b in range(nbuf):
                j = nbuf * it + b
                pltpu.make_async_copy(
                    table_hbm.at[idx_all.at[pl.ds(0, _CH)]], rows[b], gsem[b]).wait()
                pltpu.async_copy(
                    rows[b], out_hbm.at[pl.ds(base_w + j * _CH, _CH)], wsem[b])
                j2 = j + nbuf

                @pl.when(j2 < nch)
                def _():
                    pltpu.make_async_copy(
                        rows[b], out_hbm.at[pl.ds(0, _CH)], wsem[b]).wait()
                    pltpu.async_copy(
                        table_hbm.at[idx_all.at[pl.ds(j2 * _CH, _CH)]],
                        rows[b], gsem[b])
            return carry

        lax.fori_loop(0, nch // nbuf, body, 0)
        for b in range(nbuf):
            pltpu.make_async_copy(
                rows[b], out_hbm.at[pl.ds(0, _CH)], wsem[b]).wait()

    return gk(table, idx)


def _softplus(z):
    return jnp.log(1.0 + jnp.exp(-jnp.abs(z))) + jnp.maximum(z, 0.0)


def _sigmoid(z):
    return 0.5 * jnp.tanh(0.5 * z) + 0.5


_RT = 1024
_NT = 1024
_GT = _NT * M


def _embed_body(af_ref, wemb_ref, wcat_ref, x_ref, ps_ref, pn_ref):
    x = jnp.dot(af_ref[...], wemb_ref[...], preferred_element_type=jnp.float32)
    x_ref[...] = x
    p = jnp.dot(x, wcat_ref[...], preferred_element_type=jnp.float32)
    ps_ref[...] = p[:, :F2]
    pn_ref[...] = p[:, F2:]


def _embed(atom_fea, W_emb, Wcat):
    return pl.pallas_call(
        _embed_body,
        grid=(N // _RT,),
        in_specs=[
            pl.BlockSpec((_RT, ORIG), lambda i: (i, 0)),
            pl.BlockSpec((ORIG, AF), lambda i: (0, 0)),
            pl.BlockSpec((AF, 2 * F2), lambda i: (0, 0)),
        ],
        out_specs=[
            pl.BlockSpec((_RT, AF), lambda i: (i, 0)),
            pl.BlockSpec((_RT, F2), lambda i: (i, 0)),
            pl.BlockSpec((_RT, F2), lambda i: (i, 0)),
        ],
        out_shape=[
            jax.ShapeDtypeStruct((N, AF), jnp.float32),
            jax.ShapeDtypeStruct((N, F2), jnp.float32),
            jax.ShapeDtypeStruct((N, F2), jnp.float32),
        ],
    )(atom_fea, W_emb, Wcat)


def _stats_body(an_ref, nbr_ref, ps_ref, wfe_ref, g_ref, su_ref, sq_ref):
    nbr = nbr_ref[...].reshape(M * _NT, NBR)
    pe = jnp.dot(nbr, wfe_ref[...], preferred_element_type=jnp.float32)
    psr = jnp.broadcast_to(ps_ref[...][None], (M, _NT, F2)).reshape(M * _NT, F2)
    g = pe + an_ref[...].reshape(M * _NT, F2) + psr
    g_ref[...] = g.astype(jnp.bfloat16).reshape(M, _NT, F2)

    @pl.when(pl.program_id(0) == 0)
    def _():
        su_ref[...] = jnp.zeros_like(su_ref)
        sq_ref[...] = jnp.zeros_like(sq_ref)

    su_ref[...] += jnp.sum(g, axis=0, keepdims=True)
    sq_ref[...] += jnp.sum(g * g, axis=0, keepdims=True)


def _stats(an3, nbr3, ps, Wfe):
    return pl.pallas_call(
        _stats_body,
        grid=(N // _NT,),
        in_specs=[
            pl.BlockSpec((M, _NT, F2), lambda i: (0, i, 0)),
            pl.BlockSpec((M, _NT, NBR), lambda i: (0, i, 0)),
            pl.BlockSpec((_NT, F2), lambda i: (i, 0)),
            pl.BlockSpec((NBR, F2), lambda i: (0, 0)),
        ],
        out_specs=[
            pl.BlockSpec((M, _NT, F2), lambda i: (0, i, 0)),
            pl.BlockSpec((1, F2), lambda i: (0, 0)),
            pl.BlockSpec((1, F2), lambda i: (0, 0)),
        ],
        out_shape=[
            jax.ShapeDtypeStruct((M, N, F2), jnp.bfloat16),
            jax.ShapeDtypeStruct((1, F2), jnp.float32),
            jax.ShapeDtypeStruct((1, F2), jnp.float32),
        ],
    )(an3, nbr3, ps, Wfe)


def _act_body(g_ref, su_ref, sq_ref, g1_ref, bt1_ref, s_ref, ssu_ref, ssq_ref):
    mean = su_ref[...] / K
    var = sq_ref[...] / K - mean * mean
    a = g1_ref[...] * lax.rsqrt(var + 1e-5)
    c = bt1_ref[...] - mean * a
    g = g_ref[...].astype(jnp.float32).reshape(M * _NT, F2)
    gh = g * a + c
    fl = _sigmoid(gh[:, :AF])
    co = _softplus(gh[:, AF:])
    prod = (fl * co).reshape(M, _NT, AF)
    s = prod[0]
    for m in range(1, M):
        s = s + prod[m]
    s_ref[...] = s

    @pl.when(pl.program_id(0) == 0)
    def _():
        ssu_ref[...] = jnp.zeros_like(ssu_ref)
        ssq_ref[...] = jnp.zeros_like(ssq_ref)

    ssu_ref[...] += jnp.sum(s, axis=0, keepdims=True)
    ssq_ref[...] += jnp.sum(s * s, axis=0, keepdims=True)


def _act(g, su, sq, g1, bt1):
    return pl.pallas_call(
        _act_body,
        grid=(N // _NT,),
        in_specs=[
            pl.BlockSpec((M, _NT, F2), lambda i: (0, i, 0)),
            pl.BlockSpec((1, F2), lambda i: (0, 0)),
            pl.BlockSpec((1, F2), lambda i: (0, 0)),
            pl.BlockSpec((1, F2), lambda i: (0, 0)),
            pl.BlockSpec((1, F2), lambda i: (0, 0)),
        ],
        out_specs=[
            pl.BlockSpec((_NT, AF), lambda i: (i, 0)),
            pl.BlockSpec((1, AF), lambda i: (0, 0)),
            pl.BlockSpec((1, AF), lambda i: (0, 0)),
        ],
        out_shape=[
            jax.ShapeDtypeStruct((N, AF), jnp.float32),
            jax.ShapeDtypeStruct((1, AF), jnp.float32),
            jax.ShapeDtypeStruct((1, AF), jnp.float32),
        ],
    )(g, su, sq, g1, bt1)


def _bn2_update(x_ref, s_ref, ssu_ref, ssq_ref, g2_ref, bt2_ref):
    m2 = ssu_ref[...] / N
    v2 = ssq_ref[...] / N - m2 * m2
    a2 = g2_ref[...] * lax.rsqrt(v2 + 1e-5)
    c2 = bt2_ref[...] - m2 * a2
    return _softplus(x_ref[...] + s_ref[...] * a2 + c2)


def _update_body(x_ref, s_ref, ssu_ref, ssq_ref, g2_ref, bt2_ref, wcat_ref,
                 xo_ref, ps_ref, pn_ref):
    xn = _bn2_update(x_ref, s_ref, ssu_ref, ssq_ref, g2_ref, bt2_ref)
    xo_ref[...] = xn
    p = jnp.dot(xn, wcat_ref[...], preferred_element_type=jnp.float32)
    ps_ref[...] = p[:, :F2]
    pn_ref[...] = p[:, F2:]


def _update(x, s, ssu, ssq, g2, bt2, Wcat):
    return pl.pallas_call(
        _update_body,
        grid=(N // _RT,),
        in_specs=[
            pl.BlockSpec((_RT, AF), lambda i: (i, 0)),
            pl.BlockSpec((_RT, AF), lambda i: (i, 0)),
            pl.BlockSpec((1, AF), lambda i: (0, 0)),
            pl.BlockSpec((1, AF), lambda i: (0, 0)),
            pl.BlockSpec((1, AF), lambda i: (0, 0)),
            pl.BlockSpec((1, AF), lambda i: (0, 0)),
            pl.BlockSpec((AF, 2 * F2), lambda i: (0, 0)),
        ],
        out_specs=[
            pl.BlockSpec((_RT, AF), lambda i: (i, 0)),
            pl.BlockSpec((_RT, F2), lambda i: (i, 0)),
            pl.BlockSpec((_RT, F2), lambda i: (i, 0)),
        ],
        out_shape=[
            jax.ShapeDtypeStruct((N, AF), jnp.float32),
            jax.ShapeDtypeStruct((N, F2), jnp.float32),
            jax.ShapeDtypeStruct((N, F2), jnp.float32),
        ],
    )(x, s, ssu, ssq, g2, bt2, Wcat)


def _update_last_body(x_ref, s_ref, ssu_ref, ssq_ref, g2_ref, bt2_ref, xo_ref):
    xo_ref[...] = _bn2_update(x_ref, s_ref, ssu_ref, ssq_ref, g2_ref, bt2_ref)


def _update_last(x, s, ssu, ssq, g2, bt2):
    return pl.pallas_call(
        _update_last_body,
        grid=(N // _RT,),
        in_specs=[
            pl.BlockSpec((_RT, AF), lambda i: (i, 0)),
            pl.BlockSpec((_RT, AF), lambda i: (i, 0)),
            pl.BlockSpec((1, AF), lambda i: (0, 0)),
            pl.BlockSpec((1, AF), lambda i: (0, 0)),
            pl.BlockSpec((1, AF), lambda i: (0, 0)),
            pl.BlockSpec((1, AF), lambda i: (0, 0)),
        ],
        out_specs=pl.BlockSpec((_RT, AF), lambda i: (i, 0)),
        out_shape=jax.ShapeDtypeStruct((N, AF), jnp.float32),
    )(x, s, ssu, ssq, g2, bt2)


def _final_body(x_ref, w2c_ref, b2_ref, waf_ref, baf_ref,
                ep_ref, af_ref, z_ref, n_ref):
    x = x_ref[...]
    nrm = jnp.sqrt(jnp.sum(x * x, axis=1, keepdims=True))
    nd = x / jnp.maximum(nrm, 1e-12)
    n_ref[...] = nd[None]
    z_ref[...] = jnp.mean(nd, axis=0, keepdims=True)[None]
    af_ref[...] = jnp.dot(nd, waf_ref[...], preferred_element_type=jnp.float32) + baf_ref[...]
    tmp = jnp.dot(nd, w2c_ref[...], preferred_element_type=jnp.float32)
    tmp_r = jnp.concatenate([tmp[:, j * AF:(j + 1) * AF] for j in range(6)], axis=0)
    es_all = lax.dot_general(tmp_r, nd, (((1,), (1,)), ((), ())),
                             preferred_element_type=jnp.float32)
    es = [es_all[j * A:(j + 1) * A, :] + b2_ref[0, j] for j in range(6)]
    mx = es[0]
    for j in range(1, 6):
        mx = jnp.maximum(mx, es[j])
    se = jnp.exp(es[0] - mx)
    for j in range(1, 6):
        se += jnp.exp(es[j] - mx)
    off = mx + jnp.log(se)
    ep_ref[...] = jnp.concatenate([es[j] - off for j in range(6)], axis=0)[None]


def _final(x, W2c, b2, W_af, b_af):
    return pl.pallas_call(
        _final_body,
        grid=(B,),
        in_specs=[
            pl.BlockSpec((A, AF), lambda i: (i, 0)),
            pl.BlockSpec((AF, 6 * AF), lambda i: (0, 0)),
            pl.BlockSpec((1, 6), lambda i: (0, 0)),
            pl.BlockSpec((AF, ORIG), lambda i: (0, 0)),
            pl.BlockSpec((1, ORIG), lambda i: (0, 0)),
        ],
        out_specs=[
            pl.BlockSpec((1, 6 * A, A), lambda i: (i, 0, 0)),
            pl.BlockSpec((A, ORIG), lambda i: (i, 0)),
            pl.BlockSpec((1, 1, AF), lambda i: (i, 0, 0)),
            pl.BlockSpec((1, A, AF), lambda i: (i, 0, 0)),
        ],
        out_shape=[
            jax.ShapeDtypeStruct((B, 6 * A, A), jnp.float32),
            jax.ShapeDtypeStruct((N, ORIG), jnp.float32),
            jax.ShapeDtypeStruct((B, 1, AF), jnp.float32),
            jax.ShapeDtypeStruct((B, A, AF), jnp.float32),
        ],
    )(x, W2c, b2, W_af, b_af)


def kernel(atom_fea, nbr_fea, nbr_fea_idx, crystal_atom_idx, cuda_flag, W_emb,
           Wf0, bf0, g1_0, bt1_0, g2_0, bt2_0,
           Wf1, bf1, g1_1, bt1_1, g2_1, bt2_1,
           Wf2, bf2, g1_2, bt1_2, g2_2, bt2_2,
           W_bil, b_bil, W_fc1, b_fc1, W_af, b_af):
    Wf = [Wf0, Wf1, Wf2]
    g1 = [g1_0[None], g1_1[None], g1_2[None]]
    bt1 = [bt1_0[None], bt1_1[None], bt1_2[None]]
    g2 = [g2_0[None], g2_1[None], g2_2[None]]
    bt2 = [bt2_0[None], bt2_1[None], bt2_2[None]]
    Wcat = [jnp.concatenate([w[:AF], w[AF:2 * AF]], axis=1) for w in Wf]
    Wfe = [w[2 * AF:].astype(jnp.bfloat16) for w in Wf]
    idx = nbr_fea_idx.T.reshape(-1).astype(jnp.int32)
    nbr3 = nbr_fea.transpose(1, 0, 2).astype(jnp.bfloat16)

    x, ps, pn = _embed(atom_fea, W_emb, Wcat[0])
    for l in range(NC):
        an3 = _sc_gather(pn, idx).reshape(M, N, F2)
        g, su, sq = _stats(an3, nbr3, ps, Wfe[l])
        s, ssu, ssq = _act(g, su, sq, g1[l], bt1[l])
        if l + 1 < NC:
            x, ps, pn = _update(x, s, ssu, ssq, g2[l], bt2[l], Wcat[l + 1])
        else:
            x = _update_last(x, s, ssu, ssq, g2[l], bt2[l])

    W2 = jnp.einsum('kde,kj->jde', W_bil, W_fc1)
    W2c = jnp.concatenate([W2[j] for j in range(6)], axis=1)
    b2 = (b_bil @ W_fc1 + b_fc1)[None]
    epk, af, z, normed = _final(x, W2c, b2, W_af, b_af[None])
    ep = jnp.transpose(epk.reshape(B, 6, A, A), (0, 2, 3, 1)).reshape(-1, 6)
    return ep, af, z.reshape(B, AF), normed, x


# speedup vs baseline: 1.1030x; 1.0399x over previous
import functools

import jax
import jax.numpy as jnp
from jax import lax
from jax.experimental import pallas as pl
from jax.experimental.pallas import tpu as pltpu
from jax.experimental.pallas import tpu_sc as plsc

B, A, M = 64, 128, 12
ORIG, NBR, AF, NC = 92, 41, 64, 3
N = B * A
K = N * M
F2 = 2 * AF

_SC_CORES, _SC_SUBCORES = 2, 16
_NW = _SC_CORES * _SC_SUBCORES
_ROWS_PER_W = K // _NW
_CH = 128
_NCH = _ROWS_PER_W // _CH


def _sc_gather(table, idx, nrows=K):
    mesh = plsc.VectorSubcoreMesh(core_axis_name="c", subcore_axis_name="s")
    rows_per_w = nrows // _NW
    nch = rows_per_w // _CH

    nbuf = 4

    @functools.partial(
        pl.kernel, mesh=mesh,
        out_type=jax.ShapeDtypeStruct((nrows, F2), jnp.float32),
        scratch_types=[
            pltpu.VMEM((rows_per_w,), jnp.int32),
        ] + [pltpu.VMEM((_CH, F2), jnp.float32) for _ in range(4)]
          + [pltpu.SemaphoreType.DMA for _ in range(8)],
    )
    def gk(table_hbm, idx_hbm, out_hbm, idx_all,
           rows0, rows1, rows2, rows3,
           g0, g1_, g2_, g3, w0, w1, w2, w3):
        wid = lax.axis_index("s") * _SC_CORES + lax.axis_index("c")
        base_w = pl.multiple_of(wid * rows_per_w, rows_per_w)
        rows = (rows0, rows1, rows2, rows3)
        gsem = (g0, g1_, g2_, g3)
        wsem = (w0, w1, w2, w3)
        pltpu.sync_copy(idx_hbm.at[pl.ds(base_w, rows_per_w)], idx_all)
        for b in range(nbuf):
            pltpu.async_copy(
                table_hbm.at[idx_all.at[pl.ds(b * _CH, _CH)]], rows[b], gsem[b])

        def body(it, carry):
            for b in range(nbuf):
                j = nbuf * it + b
                pltpu.make_async_copy(
                    table_hbm.at[idx_all.at[pl.ds(0, _CH)]], rows[b], gsem[b]).wait()
                pltpu.async_copy(
                    rows[b], out_hbm.at[pl.ds(base_w + j * _CH, _CH)], wsem[b])
                j2 = j + nbuf

                @pl.when(j2 < nch)
                def _():
                    pltpu.make_async_copy(
                        rows[b], out_hbm.at[pl.ds(0, _CH)], wsem[b]).wait()
                    pltpu.async_copy(
                        table_hbm.at[idx_all.at[pl.ds(j2 * _CH, _CH)]],
                        rows[b], gsem[b])
            return carry

        lax.fori_loop(0, nch // nbuf, body, 0)
        for b in range(nbuf):
            pltpu.make_async_copy(
                rows[b], out_hbm.at[pl.ds(0, _CH)], wsem[b]).wait()

    return gk(table, idx)


def _softplus(z):
    return jnp.log(1.0 + jnp.exp(-jnp.abs(z))) + jnp.maximum(z, 0.0)


def _sigmoid(z):
    return 0.5 * jnp.tanh(0.5 * z) + 0.5


_RT = 1024
_NT = 1024
_GT = _NT * M


def _embed_body(af_ref, wemb_ref, wcat_ref, x_ref, ps_ref, pn_ref):
    x = jnp.dot(af_ref[...], wemb_ref[...], preferred_element_type=jnp.float32)
    x_ref[...] = x
    p = jnp.dot(x, wcat_ref[...], preferred_element_type=jnp.float32)
    ps_ref[...] = p[:, :F2]
    pn_ref[...] = p[:, F2:]


def _embed(atom_fea, W_emb, Wcat):
    return pl.pallas_call(
        _embed_body,
        grid=(N // _RT,),
        in_specs=[
            pl.BlockSpec((_RT, ORIG), lambda i: (i, 0)),
            pl.BlockSpec((ORIG, AF), lambda i: (0, 0)),
            pl.BlockSpec((AF, 2 * F2), lambda i: (0, 0)),
        ],
        out_specs=[
            pl.BlockSpec((_RT, AF), lambda i: (i, 0)),
            pl.BlockSpec((_RT, F2), lambda i: (i, 0)),
            pl.BlockSpec((_RT, F2), lambda i: (i, 0)),
        ],
        out_shape=[
            jax.ShapeDtypeStruct((N, AF), jnp.float32),
            jax.ShapeDtypeStruct((N, F2), jnp.float32),
            jax.ShapeDtypeStruct((N, F2), jnp.float32),
        ],
    )(atom_fea, W_emb, Wcat)


def _stats_body(an_ref, nbr_ref, ps_ref, wfe_ref, g_ref, su_ref, sq_ref):
    nbr = nbr_ref[...].reshape(M * _NT, NBR)
    pe = jnp.dot(nbr, wfe_ref[...], preferred_element_type=jnp.float32)
    psr = jnp.broadcast_to(ps_ref[...][None], (M, _NT, F2)).reshape(M * _NT, F2)
    g = pe + an_ref[...].reshape(M * _NT, F2) + psr
    g_ref[...] = g.astype(jnp.bfloat16).reshape(M, _NT, F2)

    @pl.when(pl.program_id(0) == 0)
    def _():
        su_ref[...] = jnp.zeros_like(su_ref)
        sq_ref[...] = jnp.zeros_like(sq_ref)

    su_ref[...] += jnp.sum(g, axis=0, keepdims=True)
    sq_ref[...] += jnp.sum(g * g, axis=0, keepdims=True)


def _stats(an3, nbr3, ps, Wfe):
    return pl.pallas_call(
        _stats_body,
        grid=(N // _NT,),
        in_specs=[
            pl.BlockSpec((M, _NT, F2), lambda i: (0, i, 0)),
            pl.BlockSpec((M, _NT, NBR), lambda i: (0, i, 0)),
            pl.BlockSpec((_NT, F2), lambda i: (i, 0)),
            pl.BlockSpec((NBR, F2), lambda i: (0, 0)),
        ],
        out_specs=[
            pl.BlockSpec((M, _NT, F2), lambda i: (0, i, 0)),
            pl.BlockSpec((1, F2), lambda i: (0, 0)),
            pl.BlockSpec((1, F2), lambda i: (0, 0)),
        ],
        out_shape=[
            jax.ShapeDtypeStruct((M, N, F2), jnp.bfloat16),
            jax.ShapeDtypeStruct((1, F2), jnp.float32),
            jax.ShapeDtypeStruct((1, F2), jnp.float32),
        ],
    )(an3, nbr3, ps, Wfe)


def _act_body(g_ref, su_ref, sq_ref, g1_ref, bt1_ref, s_ref, ssu_ref, ssq_ref):
    mean = su_ref[...] / K
    var = sq_ref[...] / K - mean * mean
    a = g1_ref[...] * lax.rsqrt(var + 1e-5)
    c = bt1_ref[...] - mean * a
    g = g_ref[...].astype(jnp.float32).reshape(M * _NT, F2)
    gh = g * a + c
    fl = _sigmoid(gh[:, :AF])
    co = _softplus(gh[:, AF:])
    prod = (fl * co).reshape(M, _NT, AF)
    s = prod[0]
    for m in range(1, M):
        s = s + prod[m]
    s_ref[...] = s

    @pl.when(pl.program_id(0) == 0)
    def _():
        ssu_ref[...] = jnp.zeros_like(ssu_ref)
        ssq_ref[...] = jnp.zeros_like(ssq_ref)

    ssu_ref[...] += jnp.sum(s, axis=0, keepdims=True)
    ssq_ref[...] += jnp.sum(s * s, axis=0, keepdims=True)


def _act(g, su, sq, g1, bt1):
    return pl.pallas_call(
        _act_body,
        grid=(N // _NT,),
        in_specs=[
            pl.BlockSpec((M, _NT, F2), lambda i: (0, i, 0)),
            pl.BlockSpec((1, F2), lambda i: (0, 0)),
            pl.BlockSpec((1, F2), lambda i: (0, 0)),
            pl.BlockSpec((1, F2), lambda i: (0, 0)),
            pl.BlockSpec((1, F2), lambda i: (0, 0)),
        ],
        out_specs=[
            pl.BlockSpec((_NT, AF), lambda i: (i, 0)),
            pl.BlockSpec((1, AF), lambda i: (0, 0)),
            pl.BlockSpec((1, AF), lambda i: (0, 0)),
        ],
        out_shape=[
            jax.ShapeDtypeStruct((N, AF), jnp.float32),
            jax.ShapeDtypeStruct((1, AF), jnp.float32),
            jax.ShapeDtypeStruct((1, AF), jnp.float32),
        ],
    )(g, su, sq, g1, bt1)


def _bn2_update(x_ref, s_ref, ssu_ref, ssq_ref, g2_ref, bt2_ref):
    m2 = ssu_ref[...] / N
    v2 = ssq_ref[...] / N - m2 * m2
    a2 = g2_ref[...] * lax.rsqrt(v2 + 1e-5)
    c2 = bt2_ref[...] - m2 * a2
    return _softplus(x_ref[...] + s_ref[...] * a2 + c2)


def _update_body(x_ref, s_ref, ssu_ref, ssq_ref, g2_ref, bt2_ref, wcat_ref,
                 xo_ref, ps_ref, pn_ref):
    xn = _bn2_update(x_ref, s_ref, ssu_ref, ssq_ref, g2_ref, bt2_ref)
    xo_ref[...] = xn
    p = jnp.dot(xn, wcat_ref[...], preferred_element_type=jnp.float32)
    ps_ref[...] = p[:, :F2]
    pn_ref[...] = p[:, F2:]


def _update(x, s, ssu, ssq, g2, bt2, Wcat):
    return pl.pallas_call(
        _update_body,
        grid=(N // _RT,),
        in_specs=[
            pl.BlockSpec((_RT, AF), lambda i: (i, 0)),
            pl.BlockSpec((_RT, AF), lambda i: (i, 0)),
            pl.BlockSpec((1, AF), lambda i: (0, 0)),
            pl.BlockSpec((1, AF), lambda i: (0, 0)),
            pl.BlockSpec((1, AF), lambda i: (0, 0)),
            pl.BlockSpec((1, AF), lambda i: (0, 0)),
            pl.BlockSpec((AF, 2 * F2), lambda i: (0, 0)),
        ],
        out_specs=[
            pl.BlockSpec((_RT, AF), lambda i: (i, 0)),
            pl.BlockSpec((_RT, F2), lambda i: (i, 0)),
            pl.BlockSpec((_RT, F2), lambda i: (i, 0)),
        ],
        out_shape=[
            jax.ShapeDtypeStruct((N, AF), jnp.float32),
            jax.ShapeDtypeStruct((N, F2), jnp.float32),
            jax.ShapeDtypeStruct((N, F2), jnp.float32),
        ],
    )(x, s, ssu, ssq, g2, bt2, Wcat)


def _update_last_body(x_ref, s_ref, ssu_ref, ssq_ref, g2_ref, bt2_ref, xo_ref):
    xo_ref[...] = _bn2_update(x_ref, s_ref, ssu_ref, ssq_ref, g2_ref, bt2_ref)


def _update_last(x, s, ssu, ssq, g2, bt2):
    return pl.pallas_call(
        _update_last_body,
        grid=(N // _RT,),
        in_specs=[
            pl.BlockSpec((_RT, AF), lambda i: (i, 0)),
            pl.BlockSpec((_RT, AF), lambda i: (i, 0)),
            pl.BlockSpec((1, AF), lambda i: (0, 0)),
            pl.BlockSpec((1, AF), lambda i: (0, 0)),
            pl.BlockSpec((1, AF), lambda i: (0, 0)),
            pl.BlockSpec((1, AF), lambda i: (0, 0)),
        ],
        out_specs=pl.BlockSpec((_RT, AF), lambda i: (i, 0)),
        out_shape=jax.ShapeDtypeStruct((N, AF), jnp.float32),
    )(x, s, ssu, ssq, g2, bt2)


_CB = 2


def _final_body(x_ref, w2c_ref, b2_ref, waf_ref, baf_ref,
                ep_ref, af_ref, z_ref, n_ref):
    x = x_ref[...]
    nrm = jnp.sqrt(jnp.sum(x * x, axis=1, keepdims=True))
    nd = x / jnp.maximum(nrm, 1e-12)
    n_ref[...] = nd.reshape(_CB, A, AF)
    af_ref[...] = jnp.dot(nd, waf_ref[...], preferred_element_type=jnp.float32) + baf_ref[...]
    tmp = jnp.dot(nd, w2c_ref[...], preferred_element_type=jnp.float32)
    for cb in range(_CB):
        ndc = nd[cb * A:(cb + 1) * A, :]
        z_ref[cb, ...] = jnp.mean(ndc, axis=0, keepdims=True)
        tmpc = tmp[cb * A:(cb + 1) * A, :]
        tmp_r = jnp.concatenate([tmpc[:, j * AF:(j + 1) * AF] for j in range(6)], axis=0)
        es_all = lax.dot_general(tmp_r, ndc, (((1,), (1,)), ((), ())),
                                 preferred_element_type=jnp.float32)
        es = [es_all[j * A:(j + 1) * A, :] + b2_ref[0, j] for j in range(6)]
        mx = es[0]
        for j in range(1, 6):
            mx = jnp.maximum(mx, es[j])
        se = jnp.exp(es[0] - mx)
        for j in range(1, 6):
            se += jnp.exp(es[j] - mx)
        off = mx + jnp.log(se)
        ep_ref[cb, ...] = jnp.concatenate([es[j] - off for j in range(6)], axis=0)


def _final(x, W2c, b2, W_af, b_af):
    return pl.pallas_call(
        _final_body,
        grid=(B // _CB,),
        in_specs=[
            pl.BlockSpec((_CB * A, AF), lambda i: (i, 0)),
            pl.BlockSpec((AF, 6 * AF), lambda i: (0, 0)),
            pl.BlockSpec((1, 6), lambda i: (0, 0)),
            pl.BlockSpec((AF, ORIG), lambda i: (0, 0)),
            pl.BlockSpec((1, ORIG), lambda i: (0, 0)),
        ],
        out_specs=[
            pl.BlockSpec((_CB, 6 * A, A), lambda i: (i, 0, 0)),
            pl.BlockSpec((_CB * A, ORIG), lambda i: (i, 0)),
            pl.BlockSpec((_CB, 1, AF), lambda i: (i, 0, 0)),
            pl.BlockSpec((_CB, A, AF), lambda i: (i, 0, 0)),
        ],
        out_shape=[
            jax.ShapeDtypeStruct((B, 6 * A, A), jnp.float32),
            jax.ShapeDtypeStruct((N, ORIG), jnp.float32),
            jax.ShapeDtypeStruct((B, 1, AF), jnp.float32),
            jax.ShapeDtypeStruct((B, A, AF), jnp.float32),
        ],
    )(x, W2c, b2, W_af, b_af)


def kernel(atom_fea, nbr_fea, nbr_fea_idx, crystal_atom_idx, cuda_flag, W_emb,
           Wf0, bf0, g1_0, bt1_0, g2_0, bt2_0,
           Wf1, bf1, g1_1, bt1_1, g2_1, bt2_1,
           Wf2, bf2, g1_2, bt1_2, g2_2, bt2_2,
           W_bil, b_bil, W_fc1, b_fc1, W_af, b_af):
    Wf = [Wf0, Wf1, Wf2]
    g1 = [g1_0[None], g1_1[None], g1_2[None]]
    bt1 = [bt1_0[None], bt1_1[None], bt1_2[None]]
    g2 = [g2_0[None], g2_1[None], g2_2[None]]
    bt2 = [bt2_0[None], bt2_1[None], bt2_2[None]]
    Wcat = [jnp.concatenate([w[:AF], w[AF:2 * AF]], axis=1) for w in Wf]
    Wfe = [w[2 * AF:].astype(jnp.bfloat16) for w in Wf]
    idx = nbr_fea_idx.T.reshape(-1).astype(jnp.int32)
    nbr3 = nbr_fea.transpose(1, 0, 2).astype(jnp.bfloat16)

    x, ps, pn = _embed(atom_fea, W_emb, Wcat[0])
    for l in range(NC):
        an3 = _sc_gather(pn, idx).reshape(M, N, F2)
        g, su, sq = _stats(an3, nbr3, ps, Wfe[l])
        s, ssu, ssq = _act(g, su, sq, g1[l], bt1[l])
        if l + 1 < NC:
            x, ps, pn = _update(x, s, ssu, ssq, g2[l], bt2[l], Wcat[l + 1])
        else:
            x = _update_last(x, s, ssu, ssq, g2[l], bt2[l])

    W2 = jnp.einsum('kde,kj->jde', W_bil, W_fc1)
    W2c = jnp.concatenate([W2[j] for j in range(6)], axis=1)
    b2 = (b_bil @ W_fc1 + b_fc1)[None]
    epk, af, z, normed = _final(x, W2c, b2, W_af, b_af[None])
    ep = jnp.transpose(epk.reshape(B, 6, A, A), (0, 2, 3, 1)).reshape(-1, 6)
    return ep, af, z.reshape(B, AF), normed, x
